# baseline (speedup 1.0000x reference)
import functools

import jax
import jax.numpy as jnp
from jax import lax
from jax.experimental import pallas as pl
from jax.experimental.pallas import tpu as pltpu
from jax.experimental.pallas import tpu_sc as plsc

_N = 10000
_E = 320000
_T = 320000
_H = 64
_LOG2 = 0.6931471805599453

_NC = 2
_NS = 16
_NW = _NC * _NS


def _ssp(x):
    return jnp.maximum(x, 0.0) + jnp.log(1.0 + jnp.exp(-jnp.abs(x))) - _LOG2


def _sigmoid(x):
    return 1.0 / (1.0 + jnp.exp(-x))


def _node_mlp(an, emb_pad, w1, b1, w2, b2):
    B = 2000

    def body(an_ref, emb_ref, w1_ref, b1_ref, w2_ref, b2_ref, out_ref):
        a = an_ref[...]
        cols = lax.broadcasted_iota(jnp.int32, (B, 128), 1)
        oh = (cols == a).astype(jnp.float32)
        x = _ssp(oh @ emb_ref[...])
        x = _ssp(x @ w1_ref[...] + b1_ref[...])
        out_ref[...] = x @ w2_ref[...] + b2_ref[...]

    return pl.pallas_call(
        body,
        grid=(_N // B,),
        in_specs=[
            pl.BlockSpec((B, 1), lambda s: (s, 0)),
            pl.BlockSpec((128, _H), lambda s: (0, 0)),
            pl.BlockSpec((_H, _H), lambda s: (0, 0)),
            pl.BlockSpec((1, _H), lambda s: (0, 0)),
            pl.BlockSpec((_H, _H), lambda s: (0, 0)),
            pl.BlockSpec((1, _H), lambda s: (0, 0)),
        ],
        out_specs=pl.BlockSpec((B, _H), lambda s: (s, 0)),
        out_shape=jax.ShapeDtypeStruct((_N, _H), jnp.float32),
    )(an, emb_pad, w1, b1, w2, b2)


def _gauss(d):
    B = 8000
    step = 5.0 / (_H - 1)
    coeff = -0.5 / (step * step)

    def body(d_ref, out_ref):
        off = lax.broadcasted_iota(jnp.int32, (B, _H), 1).astype(
            jnp.float32) * step
        z = d_ref[...] - off
        out_ref[...] = jnp.exp(coeff * z * z)

    return pl.pallas_call(
        body,
        grid=(_E // B,),
        in_specs=[pl.BlockSpec((B, 1), lambda s: (s, 0))],
        out_specs=pl.BlockSpec((B, _H), lambda s: (s, 0)),
        out_shape=jax.ShapeDtypeStruct((_E, _H), jnp.float32),
    )(d)


def _rowmm(x, w, B):
    M = x.shape[0]

    def body(x_ref, w_ref, out_ref):
        out_ref[...] = x_ref[...] @ w_ref[...]

    return pl.pallas_call(
        body,
        grid=(M // B,),
        in_specs=[
            pl.BlockSpec((B, _H), lambda s: (s, 0)),
            pl.BlockSpec((_H, 128), lambda s: (0, 0)),
        ],
        out_specs=pl.BlockSpec((B, 128), lambda s: (s, 0)),
        out_shape=jax.ShapeDtypeStruct((M, 128), jnp.float32),
    )(x, w)


def _stats_update(st_ref, y):
    @pl.when(pl.program_id(0) == 0)
    def _():
        st_ref[...] = jnp.zeros_like(st_ref)

    st_ref[0:1, :] += jnp.sum(y, axis=0, keepdims=True)
    st_ref[1:2, :] += jnp.sum(y * y, axis=0, keepdims=True)


def _c1_stats(a, x, w, b):
    B = 4000

    def body(a_ref, x_ref, w_ref, b_ref, y_ref, st_ref):
        y = a_ref[...] + x_ref[...] @ w_ref[...] + b_ref[...]
        y_ref[...] = y
        _stats_update(st_ref, y)

    return pl.pallas_call(
        body,
        grid=(_E // B,),
        in_specs=[
            pl.BlockSpec((B, 128), lambda s: (s, 0)),
            pl.BlockSpec((B, _H), lambda s: (s, 0)),
            pl.BlockSpec((_H, 128), lambda s: (0, 0)),
            pl.BlockSpec((1, 128), lambda s: (0, 0)),
        ],
        out_specs=[
            pl.BlockSpec((B, 128), lambda s: (s, 0)),
            pl.BlockSpec((8, 128), lambda s: (0, 0)),
        ],
        out_shape=[
            jax.ShapeDtypeStruct((_E, 128), jnp.float32),
            jax.ShapeDtypeStruct((8, 128), jnp.float32),
        ],
    )(a, x, w, b)


def _mul_stats(x1, x2, w, b):
    B = 4000

    def body(x1_ref, x2_ref, w_ref, b_ref, y_ref, st_ref):
        x1 = x1_ref[:, 0:_H]
        x2 = x2_ref[:, 0:_H]
        y = (x1 * x2) @ w_ref[...] + b_ref[...]
        y_ref[...] = y
        _stats_update(st_ref, y)

    return pl.pallas_call(
        body,
        grid=(_E // B,),
        in_specs=[
            pl.BlockSpec((B, 128), lambda s: (s, 0)),
            pl.BlockSpec((B, 128), lambda s: (s, 0)),
            pl.BlockSpec((_H, 128), lambda s: (0, 0)),
            pl.BlockSpec((1, 128), lambda s: (0, 0)),
        ],
        out_specs=[
            pl.BlockSpec((B, 128), lambda s: (s, 0)),
            pl.BlockSpec((8, 128), lambda s: (0, 0)),
        ],
        out_shape=[
            jax.ShapeDtypeStruct((_E, 128), jnp.float32),
            jax.ShapeDtypeStruct((8, 128), jnp.float32),
        ],
    )(x1, x2, w, b)


def _add5_stats(g1, g2, g3, g4, g5, b):
    B = 4000

    def body(r1, r2, r3, r4, r5, b_ref, y_ref, st_ref):
        y = r1[...] + r2[...] + r3[...] + r4[...] + r5[...] + b_ref[...]
        y_ref[...] = y
        _stats_update(st_ref, y)

    blk = pl.BlockSpec((B, 128), lambda s: (s, 0))
    return pl.pallas_call(
        body,
        grid=(_T // B,),
        in_specs=[blk, blk, blk, blk, blk, pl.BlockSpec((1, 128), lambda s: (0, 0))],
        out_specs=[blk, pl.BlockSpec((8, 128), lambda s: (0, 0))],
        out_shape=[
            jax.ShapeDtypeStruct((_T, 128), jnp.float32),
            jax.ShapeDtypeStruct((8, 128), jnp.float32),
        ],
    )(g1, g2, g3, g4, g5, b)


def _bn_gate(y, m, iv, g, b):
    B = 4000
    M = y.shape[0]

    def body(y_ref, m_ref, iv_ref, g_ref, b_ref, out_ref):
        z = (y_ref[...] - m_ref[...]) * iv_ref[...] * g_ref[...] + b_ref[...]
        f = z[:, 0:_H]
        c = z[:, _H:128]
        gated = _sigmoid(f) * jnp.tanh(c)
        out_ref[...] = jnp.concatenate(
            [gated, jnp.zeros((B, _H), jnp.float32)], axis=1)

    p = pl.BlockSpec((1, 128), lambda s: (0, 0))
    return pl.pallas_call(
        body,
        grid=(M // B,),
        in_specs=[pl.BlockSpec((B, 128), lambda s: (s, 0)), p, p, p, p],
        out_specs=pl.BlockSpec((B, 128), lambda s: (s, 0)),
        out_shape=jax.ShapeDtypeStruct((M, 128), jnp.float32),
    )(y, m, iv, g, b)


def _stats64(x):
    B = 8000
    M, W = x.shape

    def body(x_ref, st_ref):
        _stats_update(st_ref, x_ref[:, 0:_H])

    return pl.pallas_call(
        body,
        grid=(M // B,),
        in_specs=[pl.BlockSpec((B, W), lambda s: (s, 0))],
        out_specs=pl.BlockSpec((8, _H), lambda s: (0, 0)),
        out_shape=jax.ShapeDtypeStruct((8, _H), jnp.float32),
    )(x)


def _add_halves(x1, x2, B):
    M = x1.shape[0]

    def body(a_ref, b_ref, out_ref):
        out_ref[...] = a_ref[:, 0:_H] + b_ref[:, 0:_H]

    d = pl.BlockSpec((B, 128), lambda s: (s, 0))
    return pl.pallas_call(
        body,
        grid=(M // B,),
        in_specs=[d, d],
        out_specs=pl.BlockSpec((B, _H), lambda s: (s, 0)),
        out_shape=jax.ShapeDtypeStruct((M, _H), jnp.float32),
    )(x1, x2)


def _node_update(agg, node_emb, g, b):
    def body(a_ref, n_ref, g_ref, b_ref, out_ref):
        a = a_ref[...]
        m = jnp.mean(a, axis=0, keepdims=True)
        v = jnp.mean(a * a, axis=0, keepdims=True) - m * m
        an = (a - m) / jnp.sqrt(v + 1e-5) * g_ref[...] + b_ref[...]
        out_ref[...] = jnp.tanh(n_ref[...] + an)

    p = pl.BlockSpec((1, _H), lambda: (0, 0))
    return pl.pallas_call(
        body,
        in_specs=[
            pl.BlockSpec((_N, _H), lambda: (0, 0)),
            pl.BlockSpec((_N, _H), lambda: (0, 0)),
            p,
            p,
        ],
        out_specs=pl.BlockSpec((_N, _H), lambda: (0, 0)),
        out_shape=jax.ShapeDtypeStruct((_N, _H), jnp.float32),
    )(agg, node_emb, g, b)


def _combine(e, c2g, c3s, m22, iv22, g22, b22, m32, iv32, g32, b32):
    B = 8000

    def body(e_ref, c2_ref, c3_ref, m2r, i2r, g2r, b2r, m3r, i3r, g3r, b3r,
             out_ref):
        c2n = (c2_ref[:, 0:_H] - m2r[...]) * i2r[...] * g2r[...] + b2r[...]
        c3n = (c3_ref[...] - m3r[...]) * i3r[...] * g3r[...] + b3r[...]
        out_ref[...] = jnp.tanh(e_ref[...] + c2n + c3n)

    d = pl.BlockSpec((B, _H), lambda s: (s, 0))
    d2 = pl.BlockSpec((B, 128), lambda s: (s, 0))
    p = pl.BlockSpec((1, _H), lambda s: (0, 0))
    return pl.pallas_call(
        body,
        grid=(_E // B,),
        in_specs=[d, d2, d, p, p, p, p, p, p, p, p],
        out_specs=d,
        out_shape=jax.ShapeDtypeStruct((_E, _H), jnp.float32),
    )(e, c2g, c3s, m22, iv22, g22, b22, m32, iv32, g32, b32)


_C = 80


@functools.lru_cache(maxsize=1)
def _mesh():
    return plsc.VectorSubcoreMesh(core_axis_name="c", subcore_axis_name="s")


def _sc_gather(table, idx):
    M = idx.shape[0]
    D = table.shape[1]
    per = M // _NW
    steps = per // _C

    @functools.partial(
        pl.kernel,
        mesh=_mesh(),
        out_type=jax.ShapeDtypeStruct((M, D), jnp.float32),
        scratch_types=[
            pltpu.VMEM((_C,), jnp.int32),
            pltpu.VMEM((_C, D), jnp.float32),
            pltpu.SemaphoreType.DMA,
        ],
    )
    def k(table_hbm, idx_hbm, out_hbm, idx_v, rows_v, sem):
        wid = lax.axis_index("s") * _NC + lax.axis_index("c")
        base = wid * per

        def body(s, carry):
            o = base + s * _C
            pltpu.sync_copy(idx_hbm.at[pl.ds(o, _C)], idx_v)
            pltpu.async_copy(table_hbm.at[idx_v], rows_v, sem).wait()
            pltpu.sync_copy(rows_v, out_hbm.at[pl.ds(o, _C)])
            return carry

        lax.fori_loop(0, steps, body, 0)

    return k(table, idx)


def _sc_segsum(data, idx, ch, npass, cc):
    M = data.shape[0]
    D = data.shape[1]
    per = M // _NW
    steps = per // _C
    rpt = ch // _NS
    reps = rpt // cc

    @functools.partial(
        pl.kernel,
        mesh=_mesh(),
        out_type=jax.ShapeDtypeStruct((2 * npass * ch, D), jnp.float32),
        scratch_types=[
            pltpu.VMEM((_C,), jnp.int32),
            pltpu.VMEM((_C,), jnp.int32),
            pltpu.VMEM((_C, D), jnp.float32),
            pltpu.VMEM((cc, D), jnp.float32),
            pltpu.VMEM((cc, D), jnp.float32),
            pltpu.VMEM_SHARED((ch + 8, D), jnp.float32),
            pltpu.SemaphoreType.DMA,
        ],
    )
    def k(data_hbm, idx_hbm, out_hbm, idx_v, loc_v, dat_v, cbuf, zbuf,
          shared, sem):
        cid = lax.axis_index("c")
        sid = lax.axis_index("s")
        wid = sid * _NC + cid
        base = wid * per

        def zrow(r, carry):
            z16 = jnp.zeros((16,), jnp.float32)
            for cq in range(D // 16):
                zbuf[r, pl.ds(cq * 16, 16)] = z16
            return carry

        lax.fori_loop(0, cc, zrow, 0)

        for p in range(npass):
            chunk = p * ch

            for q in range(reps):
                pltpu.sync_copy(zbuf, shared.at[pl.ds(sid * rpt + q * cc, cc)])

            @pl.when(sid == 0)
            def _():
                pltpu.sync_copy(zbuf.at[pl.ds(0, 8)], shared.at[pl.ds(ch, 8)])

            plsc.subcore_barrier()

            def body(s, carry):
                o = base + s * _C
                pltpu.sync_copy(idx_hbm.at[pl.ds(o, _C)], idx_v)
                pltpu.sync_copy(data_hbm.at[pl.ds(o, _C)], dat_v)

                def inner(kk, c2):
                    v = idx_v[pl.ds(kk * 16, 16)]
                    li = v - chunk
                    oob = (li < 0) | (li >= ch)
                    loc_v[pl.ds(kk * 16, 16)] = jnp.where(oob, ch, li)
                    return c2

                lax.fori_loop(0, _C // 16, inner, 0)
                pltpu.sync_copy(dat_v, shared.at[loc_v], add=True)
                return carry

            lax.fori_loop(0, steps, body, 0)
            plsc.subcore_barrier()

            for q in range(reps):
                off = sid * rpt + q * cc
                pltpu.sync_copy(shared.at[pl.ds(off, cc)], cbuf)
                pltpu.sync_copy(
                    cbuf,
                    out_hbm.at[pl.ds(cid * (npass * ch) + chunk + off, cc)])
            plsc.subcore_barrier()

    return k(data, idx)


def _finalize(st, n):
    m = st[0] / n
    var = st[1] / n - m * m
    iv = 1.0 / jnp.sqrt(var + 1e-5)
    return m.reshape(1, -1), iv.reshape(1, -1)


def kernel(atomic_numbers, distances, i, j, idx_i, idx_j, idx_k, idx_ji,
           idx_kj, emb, ne_W1, ne_b1, ne_W2, ne_b2, nb_lin_W, nb_lin_b,
           nb_bn1_g, nb_bn1_b, nb_bn2_g, nb_bn2_b, eb_lin2_W, eb_lin2_b,
           eb_lin3_W, eb_lin3_b, eb_bn2_g, eb_bn2_b, eb_bn3_g, eb_bn3_b,
           eb_bn22_g, eb_bn22_b, eb_bn32_g, eb_bn32_b):
    f32 = jnp.float32
    an = atomic_numbers.astype(jnp.int32).reshape(_N, 1)
    emb_pad = jnp.zeros((128, _H), f32).at[:95].set(emb.astype(f32))
    node_emb = _node_mlp(an, emb_pad, ne_W1, ne_b1.reshape(1, -1), ne_W2,
                         ne_b2.reshape(1, -1))
    edge_emb = _gauss(distances.astype(f32).reshape(_E, 1))

    i32 = i.astype(jnp.int32)
    j32 = j.astype(jnp.int32)
    ii32 = idx_i.astype(jnp.int32)
    ij32 = idx_j.astype(jnp.int32)
    ik32 = idx_k.astype(jnp.int32)
    iji32 = idx_ji.astype(jnp.int32)
    ikj32 = idx_kj.astype(jnp.int32)

    for l in range(2):
        wn = nb_lin_W[l][:_H]
        we = nb_lin_W[l][_H:]
        pn = _rowmm(node_emb, wn, 2000)
        png = _sc_gather(pn, i32)
        c1, st1 = _c1_stats(png, edge_emb, we, nb_lin_b[l].reshape(1, -1))
        m1, iv1 = _finalize(st1, _E)
        msg = _bn_gate(c1, m1, iv1, nb_bn1_g[l].reshape(1, -1),
                       nb_bn1_b[l].reshape(1, -1))
        aggp = _sc_segsum(msg, i32, 2048, 5, 128)
        agg = _add_halves(aggp[:10240], aggp[10240:], 2048)
        node_emb = _node_update(agg[:_N], node_emb,
                                nb_bn2_g[l].reshape(1, -1),
                                nb_bn2_b[l].reshape(1, -1))

        nep = jnp.concatenate([node_emb, jnp.zeros((_N, _H), f32)], axis=1)
        gi = _sc_gather(nep, i32)
        gj = _sc_gather(nep, j32)
        c2, st2 = _mul_stats(gi, gj, eb_lin2_W[l], eb_lin2_b[l].reshape(1, -1))
        m2, iv2 = _finalize(st2, _E)
        c2g = _bn_gate(c2, m2, iv2, eb_bn2_g[l].reshape(1, -1),
                       eb_bn2_b[l].reshape(1, -1))
        st22 = _stats64(c2g)
        m22, iv22 = _finalize(st22, _E)

        w3 = eb_lin3_W[l]
        pi = _rowmm(node_emb, w3[0:_H], 2000)
        pj = _rowmm(node_emb, w3[_H:2 * _H], 2000)
        pk = _rowmm(node_emb, w3[2 * _H:3 * _H], 2000)
        qji = _rowmm(edge_emb, w3[3 * _H:4 * _H], 8000)
        qkj = _rowmm(edge_emb, w3[4 * _H:5 * _H], 8000)
        g1 = _sc_gather(pi, ii32)
        g2 = _sc_gather(pj, ij32)
        g3 = _sc_gather(pk, ik32)
        g4 = _sc_gather(qji, iji32)
        g5 = _sc_gather(qkj, ikj32)
        c3, st3 = _add5_stats(g1, g2, g3, g4, g5,
                              eb_lin3_b[l].reshape(1, -1))
        m3, iv3 = _finalize(st3, _T)
        c3m = _bn_gate(c3, m3, iv3, eb_bn3_g[l].reshape(1, -1),
                       eb_bn3_b[l].reshape(1, -1))
        c3sp = _sc_segsum(c3m, iji32, 4992, 65, 312)
        c3sum = _add_halves(c3sp[:324480], c3sp[324480:], 4160)
        c3s = c3sum[:_E]
        st32 = _stats64(c3s)
        m32, iv32 = _finalize(st32, _E)

        edge_emb = _combine(edge_emb, c2g, c3s,
                            m22, iv22, eb_bn22_g[l].reshape(1, -1),
                            eb_bn22_b[l].reshape(1, -1),
                            m32, iv32, eb_bn32_g[l].reshape(1, -1),
                            eb_bn32_b[l].reshape(1, -1))
    return edge_emb
